# baseline (speedup 1.0000x reference)
import functools

import jax
import jax.numpy as jnp
from jax import lax
from jax.experimental import pallas as pl
from jax.experimental.pallas import tpu as pltpu
from jax.experimental.pallas import tpu_sc as plsc

B = 16384
L = 50
D = 32
P = 4
DP = D * P


_NW = 32
_UV_ROWS = (L * B) // 128
_UV_ROWS_PER_W = _UV_ROWS // _NW
_CH = 8
_NCH = _UV_ROWS_PER_W // _CH
_ND_WORKERS = 16
_ND_ROWS_PER_W = (B // 128) // _ND_WORKERS


def _sc_gather(features, idx_uv2, nodes2):
    mesh = plsc.VectorSubcoreMesh(core_axis_name="c", subcore_axis_name="s")

    @functools.partial(
        pl.kernel,
        mesh=mesh,
        out_type=(
            jax.ShapeDtypeStruct((L * B, D), jnp.float32),
            jax.ShapeDtypeStruct((B, D), jnp.float32),
        ),
        scratch_types=[
            pltpu.VMEM((_CH, 128), jnp.int32),
            pltpu.VMEM((_CH * 128, D), jnp.float32),
            pltpu.SemaphoreType.DMA,
        ],
        compiler_params=pltpu.CompilerParams(use_tc_tiling_on_sc=False, skip_device_barrier=True),
    )
    def k(feat_hbm, idx_hbm, nd_hbm, oute_hbm, outu_hbm, idx_v, rows_v, sem):
        wid = lax.axis_index("s") * 2 + lax.axis_index("c")

        def chunk(c, carry):
            r0 = wid * _UV_ROWS_PER_W + c * _CH
            pltpu.sync_copy(idx_hbm.at[pl.ds(r0, _CH)], idx_v)
            handles = []
            for j in range(_CH):
                handles.append(
                    pltpu.async_copy(
                        feat_hbm.at[idx_v.at[j]],
                        rows_v.at[pl.ds(j * 128, 128)],
                        sem,
                    )
                )
            for h in handles:
                h.wait()
            pltpu.sync_copy(rows_v, oute_hbm.at[pl.ds(r0 * 128, _CH * 128)])
            return carry

        lax.fori_loop(0, _NCH, chunk, 0)

        @pl.when(wid < _ND_WORKERS)
        def _():
            n0 = wid * _ND_ROWS_PER_W
            pltpu.sync_copy(nd_hbm.at[pl.ds(n0, _ND_ROWS_PER_W)],
                            idx_v.at[pl.ds(0, _ND_ROWS_PER_W)])
            handles = []
            for j in range(_ND_ROWS_PER_W):
                handles.append(
                    pltpu.async_copy(
                        feat_hbm.at[idx_v.at[j]],
                        rows_v.at[pl.ds(j * 128, 128)],
                        sem,
                    )
                )
            for h in handles:
                h.wait()
            pltpu.sync_copy(rows_v.at[pl.ds(0, _ND_ROWS_PER_W * 128)],
                            outu_hbm.at[pl.ds(n0 * 128, _ND_ROWS_PER_W * 128)])

    return k(features, idx_uv2, nodes2)


_BP = B // P


def _dgt(a, b):
    return jax.lax.dot_general(a, b, (((1,), (1,)), ((), ())),
                               preferred_element_type=jnp.float32)


def _tc_body(e_ref, oh_ref, u_ref, wuv4_ref, wrr4_ref, r2e4_ref, a14_ref,
             a24_ref, l14_ref, l24_ref, att24_ref, k4_ref, br4_ref,
             ba14_ref, bl14_ref, o_ref, acc, mstat, dstat):
    l = pl.program_id(0)
    e = e_ref[...]
    u = u_ref[...]
    oh = oh_ref[...]

    t2n4 = _dgt(wrr4_ref[...], r2e4_ref[...])
    xt = jnp.maximum(
        _dgt(wuv4_ref[...], e)
        + _dgt(t2n4.astype(jnp.bfloat16), oh) + br4_ref[...], 0.0)
    uat = _dgt(a24_ref[...], u)
    at = jnp.maximum(jnp.dot(a14_ref[...], xt) + uat + ba14_ref[...], 0.0)
    st = jnp.dot(att24_ref[...], at)

    @pl.when(l == 0)
    def _():
        mstat[...] = jnp.full_like(mstat, -1e30)
        dstat[...] = jnp.zeros_like(dstat)
        acc[...] = jnp.zeros_like(acc)

    m_prev = mstat[...]
    m_new = jnp.maximum(m_prev, st)
    alpha = jnp.exp(m_prev - m_new)
    p = jnp.exp(st - m_new)
    mstat[...] = m_new
    d_new = dstat[...] * alpha + p
    dstat[...] = d_new
    alpha_x = jnp.dot(k4_ref[...], alpha)
    p_x = jnp.dot(k4_ref[...], p)
    acc_new = acc[...] * alpha_x + p_x * xt
    acc[...] = acc_new

    @pl.when(l == L - 1)
    def _():
        dn_x = jnp.dot(k4_ref[...], 1.0 / d_new)
        neigh = acc_new * dn_x
        out_t = jnp.maximum(
            _dgt(l14_ref[...], u) + jnp.dot(l24_ref[...], neigh)
            + bl14_ref[...], 0.0)
        o_ref[...] = out_t.T


def _tc_fused(E, oh, U, *weights):
    full = lambda arr: pl.BlockSpec(arr.shape, lambda l: (0,) * arr.ndim)
    return pl.pallas_call(
        _tc_body,
        grid=(L,),
        in_specs=[
            pl.BlockSpec((_BP, DP), lambda l: (l, 0)),
            pl.BlockSpec((_BP, DP), lambda l: (l, 0)),
            pl.BlockSpec((_BP, DP), lambda l: (0, 0)),
        ] + [full(w) for w in weights],
        out_specs=pl.BlockSpec((_BP, DP), lambda l: (0, 0)),
        out_shape=jax.ShapeDtypeStruct((_BP, DP), jnp.float32),
        scratch_shapes=[
            pltpu.VMEM((DP, _BP), jnp.float32),
            pltpu.VMEM((P, _BP), jnp.float32),
            pltpu.VMEM((P, _BP), jnp.float32),
        ],
        compiler_params=pltpu.CompilerParams(skip_device_barrier=True),
    )(E, oh, U, *weights)


def kernel(nodes, history_uv, history_r, history_ut, features, r2e,
           W_r_w, W_r_b, att1_w, att1_b, att2_w, att2_b, lin1_w, lin1_b):
    del history_ut, att2_b
    nodes = nodes.astype(jnp.int32)
    idx_uv2 = history_uv.T.astype(jnp.int32).reshape(_UV_ROWS, 128)
    nodes2 = nodes.reshape(B // 128, 128)

    E, U = _sc_gather(features, idx_uv2, nodes2)
    E_pk = E.reshape(L * B // P, DP)
    U_pk = U.reshape(_BP, DP)

    oh = (history_r.T.reshape(L * B // P, P, 1) ==
          jnp.arange(D, dtype=history_r.dtype)
          ).reshape(L * B // P, DP).astype(jnp.bfloat16)

    eye4 = jnp.eye(P, dtype=jnp.float32)
    r2e32 = jnp.zeros((D, D), jnp.float32).at[:r2e.shape[0]].set(r2e)
    wuv4 = jnp.kron(eye4, W_r_w[:, :D])
    wrr4 = jnp.kron(eye4, W_r_w[:, D:])
    r2e4 = jnp.kron(eye4, r2e32)
    a14 = jnp.kron(eye4, att1_w[:, :D])
    a24 = jnp.kron(eye4, att1_w[:, D:])
    l14 = jnp.kron(eye4, lin1_w[:, :D])
    l24 = jnp.kron(eye4, lin1_w[:, D:])
    att24 = jnp.kron(eye4, att2_w)
    k4 = jnp.kron(eye4, jnp.ones((D, 1), jnp.float32))
    br4 = jnp.tile(W_r_b, P).reshape(DP, 1)
    ba14 = jnp.tile(att1_b, P).reshape(DP, 1)
    bl14 = jnp.tile(lin1_b, P).reshape(DP, 1)

    out_pk = _tc_fused(E_pk, oh, U_pk, wuv4, wrr4, r2e4, a14, a24,
                       l14, l24, att24, k4, br4, ba14, bl14)
    return out_pk.reshape(B, D)


# speedup vs baseline: 1.0095x; 1.0095x over previous
import functools

import jax
import jax.numpy as jnp
from jax import lax
from jax.experimental import pallas as pl
from jax.experimental.pallas import tpu as pltpu
from jax.experimental.pallas import tpu_sc as plsc

B = 16384
L = 50
D = 32
P = 4
DP = D * P


_NW = 32
_UV_ROWS = (L * B) // 128
_UV_ROWS_PER_W = _UV_ROWS // _NW
_CH = 8
_NCH = _UV_ROWS_PER_W // _CH
_ND_WORKERS = 16
_ND_ROWS_PER_W = (B // 128) // _ND_WORKERS


def _sc_gather(features, idx_uv2, nodes2):
    mesh = plsc.VectorSubcoreMesh(core_axis_name="c", subcore_axis_name="s")

    @functools.partial(
        pl.kernel,
        mesh=mesh,
        out_type=(
            jax.ShapeDtypeStruct((L * B, D), jnp.float32),
            jax.ShapeDtypeStruct((B, D), jnp.float32),
        ),
        scratch_types=[
            pltpu.VMEM((_CH, 128), jnp.int32),
            pltpu.VMEM((_CH, 128), jnp.int32),
            pltpu.VMEM((_CH * 128, D), jnp.float32),
            pltpu.VMEM((_CH * 128, D), jnp.float32),
            pltpu.SemaphoreType.DMA,
            pltpu.SemaphoreType.DMA,
        ],
        compiler_params=pltpu.CompilerParams(use_tc_tiling_on_sc=False, skip_device_barrier=True),
    )
    def k(feat_hbm, idx_hbm, nd_hbm, oute_hbm, outu_hbm,
          idx_v, idx_v2, rows_v, rows_v2, sem, semw):
        wid = lax.axis_index("s") * 2 + lax.axis_index("c")

        def gather_chunk(c, idxbuf, rowbuf):
            r0 = wid * _UV_ROWS_PER_W + c * _CH
            pltpu.sync_copy(idx_hbm.at[pl.ds(r0, _CH)], idxbuf)
            handles = []
            for j in range(_CH):
                handles.append(
                    pltpu.async_copy(
                        feat_hbm.at[idxbuf.at[j]],
                        rowbuf.at[pl.ds(j * 128, 128)],
                        sem,
                    )
                )
            for h in handles:
                h.wait()
            return pltpu.async_copy(
                rowbuf, oute_hbm.at[pl.ds(r0 * 128, _CH * 128)], semw)

        def pair(p, carry):
            wa = gather_chunk(2 * p, idx_v, rows_v)
            wb = gather_chunk(2 * p + 1, idx_v2, rows_v2)
            wa.wait()
            wb.wait()
            return carry

        lax.fori_loop(0, (_NCH - 1) // 2, pair, 0)
        gather_chunk(_NCH - 1, idx_v, rows_v).wait()

        @pl.when(wid < _ND_WORKERS)
        def _():
            n0 = wid * _ND_ROWS_PER_W
            pltpu.sync_copy(nd_hbm.at[pl.ds(n0, _ND_ROWS_PER_W)],
                            idx_v.at[pl.ds(0, _ND_ROWS_PER_W)])
            handles = []
            for j in range(_ND_ROWS_PER_W):
                handles.append(
                    pltpu.async_copy(
                        feat_hbm.at[idx_v.at[j]],
                        rows_v.at[pl.ds(j * 128, 128)],
                        sem,
                    )
                )
            for h in handles:
                h.wait()
            pltpu.sync_copy(rows_v.at[pl.ds(0, _ND_ROWS_PER_W * 128)],
                            outu_hbm.at[pl.ds(n0 * 128, _ND_ROWS_PER_W * 128)])

    return k(features, idx_uv2, nodes2)


_BP = B // P


def _dgt(a, b):
    return jax.lax.dot_general(a, b, (((1,), (1,)), ((), ())),
                               preferred_element_type=jnp.float32)


def _tc_body(e_ref, oh_ref, u_ref, wuv4_ref, wrr4_ref, r2e4_ref, a14_ref,
             a24_ref, l14_ref, l24_ref, att24_ref, k4_ref, br4_ref,
             ba14_ref, bl14_ref, o_ref, acc, mstat, dstat):
    l = pl.program_id(0)
    e = e_ref[...]
    u = u_ref[...]
    oh = oh_ref[...]

    t2n4 = _dgt(wrr4_ref[...], r2e4_ref[...])
    xt = jnp.maximum(
        _dgt(wuv4_ref[...], e)
        + _dgt(t2n4.astype(jnp.bfloat16), oh) + br4_ref[...], 0.0)
    uat = _dgt(a24_ref[...], u)
    at = jnp.maximum(jnp.dot(a14_ref[...], xt) + uat + ba14_ref[...], 0.0)
    st = jnp.dot(att24_ref[...], at)

    @pl.when(l == 0)
    def _():
        mstat[...] = jnp.full_like(mstat, -1e30)
        dstat[...] = jnp.zeros_like(dstat)
        acc[...] = jnp.zeros_like(acc)

    m_prev = mstat[...]
    m_new = jnp.maximum(m_prev, st)
    alpha = jnp.exp(m_prev - m_new)
    p = jnp.exp(st - m_new)
    mstat[...] = m_new
    d_new = dstat[...] * alpha + p
    dstat[...] = d_new
    alpha_x = jnp.dot(k4_ref[...], alpha)
    p_x = jnp.dot(k4_ref[...], p)
    acc_new = acc[...] * alpha_x + p_x * xt
    acc[...] = acc_new

    @pl.when(l == L - 1)
    def _():
        dn_x = jnp.dot(k4_ref[...], 1.0 / d_new)
        neigh = acc_new * dn_x
        out_t = jnp.maximum(
            _dgt(l14_ref[...], u) + jnp.dot(l24_ref[...], neigh)
            + bl14_ref[...], 0.0)
        o_ref[...] = out_t.T


def _tc_fused(E, oh, U, *weights):
    full = lambda arr: pl.BlockSpec(arr.shape, lambda l: (0,) * arr.ndim)
    return pl.pallas_call(
        _tc_body,
        grid=(L,),
        in_specs=[
            pl.BlockSpec((_BP, DP), lambda l: (l, 0)),
            pl.BlockSpec((_BP, DP), lambda l: (l, 0)),
            pl.BlockSpec((_BP, DP), lambda l: (0, 0)),
        ] + [full(w) for w in weights],
        out_specs=pl.BlockSpec((_BP, DP), lambda l: (0, 0)),
        out_shape=jax.ShapeDtypeStruct((_BP, DP), jnp.float32),
        scratch_shapes=[
            pltpu.VMEM((DP, _BP), jnp.float32),
            pltpu.VMEM((P, _BP), jnp.float32),
            pltpu.VMEM((P, _BP), jnp.float32),
        ],
        compiler_params=pltpu.CompilerParams(skip_device_barrier=True),
    )(E, oh, U, *weights)


def kernel(nodes, history_uv, history_r, history_ut, features, r2e,
           W_r_w, W_r_b, att1_w, att1_b, att2_w, att2_b, lin1_w, lin1_b):
    del history_ut, att2_b
    nodes = nodes.astype(jnp.int32)
    idx_uv2 = history_uv.T.astype(jnp.int32).reshape(_UV_ROWS, 128)
    nodes2 = nodes.reshape(B // 128, 128)

    E, U = _sc_gather(features, idx_uv2, nodes2)
    E_pk = E.reshape(L * B // P, DP)
    U_pk = U.reshape(_BP, DP)

    oh = (history_r.T.reshape(L * B // P, P, 1) ==
          jnp.arange(D, dtype=history_r.dtype)
          ).reshape(L * B // P, DP).astype(jnp.bfloat16)

    eye4 = jnp.eye(P, dtype=jnp.float32)
    r2e32 = jnp.zeros((D, D), jnp.float32).at[:r2e.shape[0]].set(r2e)
    wuv4 = jnp.kron(eye4, W_r_w[:, :D])
    wrr4 = jnp.kron(eye4, W_r_w[:, D:])
    r2e4 = jnp.kron(eye4, r2e32)
    a14 = jnp.kron(eye4, att1_w[:, :D])
    a24 = jnp.kron(eye4, att1_w[:, D:])
    l14 = jnp.kron(eye4, lin1_w[:, :D])
    l24 = jnp.kron(eye4, lin1_w[:, D:])
    att24 = jnp.kron(eye4, att2_w)
    k4 = jnp.kron(eye4, jnp.ones((D, 1), jnp.float32))
    br4 = jnp.tile(W_r_b, P).reshape(DP, 1)
    ba14 = jnp.tile(att1_b, P).reshape(DP, 1)
    bl14 = jnp.tile(lin1_b, P).reshape(DP, 1)

    out_pk = _tc_fused(E_pk, oh, U_pk, wuv4, wrr4, r2e4, a14, a24,
                       l14, l24, att24, k4, br4, ba14, bl14)
    return out_pk.reshape(B, D)


# speedup vs baseline: 1.0534x; 1.0435x over previous
import functools

import jax
import jax.numpy as jnp
from jax import lax
from jax.experimental import pallas as pl
from jax.experimental.pallas import tpu as pltpu
from jax.experimental.pallas import tpu_sc as plsc

B = 16384
L = 50
D = 32
P = 4
DP = D * P


_NW = 32
_UV_ROWS = (L * B) // 128
_UV_ROWS_PER_W = _UV_ROWS // _NW
_CH = 8
_NCH = _UV_ROWS_PER_W // _CH
_ND_WORKERS = 16
_ND_ROWS_PER_W = (B // 128) // _ND_WORKERS


def _sc_gather(features, idx_uv2, nodes2):
    mesh = plsc.VectorSubcoreMesh(core_axis_name="c", subcore_axis_name="s")

    @functools.partial(
        pl.kernel,
        mesh=mesh,
        out_type=(
            jax.ShapeDtypeStruct((L * B, D), jnp.float32),
            jax.ShapeDtypeStruct((B, D), jnp.float32),
        ),
        scratch_types=[
            pltpu.VMEM((_CH, 128), jnp.int32),
            pltpu.VMEM((_CH, 128), jnp.int32),
            pltpu.VMEM((_CH * 128, D), jnp.float32),
            pltpu.VMEM((_CH * 128, D), jnp.float32),
            pltpu.SemaphoreType.DMA,
            pltpu.SemaphoreType.DMA,
        ],
        compiler_params=pltpu.CompilerParams(use_tc_tiling_on_sc=False, skip_device_barrier=True),
    )
    def k(feat_hbm, idx_hbm, nd_hbm, oute_hbm, outu_hbm,
          idx_v, idx_v2, rows_v, rows_v2, sem, semw):
        wid = lax.axis_index("s") * 2 + lax.axis_index("c")

        def gather_chunk(c, idxbuf, rowbuf):
            r0 = wid * _UV_ROWS_PER_W + c * _CH
            pltpu.sync_copy(idx_hbm.at[pl.ds(r0, _CH)], idxbuf)
            handles = []
            for j in range(_CH):
                handles.append(
                    pltpu.async_copy(
                        feat_hbm.at[idxbuf.at[j]],
                        rowbuf.at[pl.ds(j * 128, 128)],
                        sem,
                    )
                )
            for h in handles:
                h.wait()
            return pltpu.async_copy(
                rowbuf, oute_hbm.at[pl.ds(r0 * 128, _CH * 128)], semw)

        def pair(p, carry):
            wa = gather_chunk(2 * p, idx_v, rows_v)
            wb = gather_chunk(2 * p + 1, idx_v2, rows_v2)
            wa.wait()
            wb.wait()
            return carry

        lax.fori_loop(0, (_NCH - 1) // 2, pair, 0)
        gather_chunk(_NCH - 1, idx_v, rows_v).wait()

        @pl.when(wid < _ND_WORKERS)
        def _():
            n0 = wid * _ND_ROWS_PER_W
            pltpu.sync_copy(nd_hbm.at[pl.ds(n0, _ND_ROWS_PER_W)],
                            idx_v.at[pl.ds(0, _ND_ROWS_PER_W)])
            handles = []
            for j in range(_ND_ROWS_PER_W):
                handles.append(
                    pltpu.async_copy(
                        feat_hbm.at[idx_v.at[j]],
                        rows_v.at[pl.ds(j * 128, 128)],
                        sem,
                    )
                )
            for h in handles:
                h.wait()
            pltpu.sync_copy(rows_v.at[pl.ds(0, _ND_ROWS_PER_W * 128)],
                            outu_hbm.at[pl.ds(n0 * 128, _ND_ROWS_PER_W * 128)])

    return k(features, idx_uv2, nodes2)


_BP = B // P


def _dgt(a, b):
    return jax.lax.dot_general(a, b, (((1,), (1,)), ((), ())),
                               preferred_element_type=jnp.float32)


def _tc_body(e_ref, oh_ref, u_ref, wuv4_ref, wrr_ref, r2e32_ref, a14_ref,
             a24_ref, l14_ref, l24_ref, att24_ref, k4_ref, br4_ref,
             ba14_ref, bl14_ref, o_ref, acc, mstat, dstat):
    l = pl.program_id(0)
    e = e_ref[...]
    u = u_ref[...]
    oh = oh_ref[...]

    t2n = _dgt(wrr_ref[...], r2e32_ref[...])
    t2n_b = jnp.broadcast_to(t2n.astype(jnp.bfloat16)[None], (P, D, D))
    rterm = jax.lax.dot_general(
        t2n_b, oh, (((2,), (1,)), ((0,), (0,))),
        preferred_element_type=jnp.float32)
    xt = jnp.maximum(
        _dgt(wuv4_ref[...], e)
        + rterm.reshape(DP, _BP) + br4_ref[...], 0.0)
    uat = _dgt(a24_ref[...], u)
    at = jnp.maximum(jnp.dot(a14_ref[...], xt) + uat + ba14_ref[...], 0.0)
    st = jnp.dot(att24_ref[...], at)

    @pl.when(l == 0)
    def _():
        mstat[...] = jnp.full_like(mstat, -1e30)
        dstat[...] = jnp.zeros_like(dstat)
        acc[...] = jnp.zeros_like(acc)

    m_prev = mstat[...]
    m_new = jnp.maximum(m_prev, st)
    alpha = jnp.exp(m_prev - m_new)
    p = jnp.exp(st - m_new)
    mstat[...] = m_new
    d_new = dstat[...] * alpha + p
    dstat[...] = d_new
    alpha_x = jnp.dot(k4_ref[...], alpha)
    p_x = jnp.dot(k4_ref[...], p)
    acc_new = acc[...] * alpha_x + p_x * xt
    acc[...] = acc_new

    @pl.when(l == L - 1)
    def _():
        dn_x = jnp.dot(k4_ref[...], 1.0 / d_new)
        neigh = acc_new * dn_x
        out_t = jnp.maximum(
            _dgt(l14_ref[...], u) + jnp.dot(l24_ref[...], neigh)
            + bl14_ref[...], 0.0)
        o_ref[...] = out_t.T


def _tc_fused(E, oh, U, *weights):
    full = lambda arr: pl.BlockSpec(arr.shape, lambda l: (0,) * arr.ndim)
    return pl.pallas_call(
        _tc_body,
        grid=(L,),
        in_specs=[
            pl.BlockSpec((_BP, DP), lambda l: (l, 0)),
            pl.BlockSpec((P, D, _BP), lambda l: (0, 0, l)),
            pl.BlockSpec((_BP, DP), lambda l: (0, 0)),
        ] + [full(w) for w in weights],
        out_specs=pl.BlockSpec((_BP, DP), lambda l: (0, 0)),
        out_shape=jax.ShapeDtypeStruct((_BP, DP), jnp.float32),
        scratch_shapes=[
            pltpu.VMEM((DP, _BP), jnp.float32),
            pltpu.VMEM((P, _BP), jnp.float32),
            pltpu.VMEM((P, _BP), jnp.float32),
        ],
        compiler_params=pltpu.CompilerParams(skip_device_barrier=True),
    )(E, oh, U, *weights)


def kernel(nodes, history_uv, history_r, history_ut, features, r2e,
           W_r_w, W_r_b, att1_w, att1_b, att2_w, att2_b, lin1_w, lin1_b):
    del history_ut, att2_b
    nodes = nodes.astype(jnp.int32)
    idx_uv2 = history_uv.T.astype(jnp.int32).reshape(_UV_ROWS, 128)
    nodes2 = nodes.reshape(B // 128, 128)

    E, U = _sc_gather(features, idx_uv2, nodes2)
    E_pk = E.reshape(L * B // P, DP)
    U_pk = U.reshape(_BP, DP)

    rT2T = history_r.T.reshape(L * B // P, P).T
    oh = (jnp.arange(D, dtype=history_r.dtype)[None, :, None] ==
          rT2T[:, None, :]).astype(jnp.bfloat16)

    eye4 = jnp.eye(P, dtype=jnp.float32)
    r2e32 = jnp.zeros((D, D), jnp.float32).at[:r2e.shape[0]].set(r2e)
    wuv4 = jnp.kron(eye4, W_r_w[:, :D])
    wrr = W_r_w[:, D:]
    a14 = jnp.kron(eye4, att1_w[:, :D])
    a24 = jnp.kron(eye4, att1_w[:, D:])
    l14 = jnp.kron(eye4, lin1_w[:, :D])
    l24 = jnp.kron(eye4, lin1_w[:, D:])
    att24 = jnp.kron(eye4, att2_w)
    k4 = jnp.kron(eye4, jnp.ones((D, 1), jnp.float32))
    br4 = jnp.tile(W_r_b, P).reshape(DP, 1)
    ba14 = jnp.tile(att1_b, P).reshape(DP, 1)
    bl14 = jnp.tile(lin1_b, P).reshape(DP, 1)

    out_pk = _tc_fused(E_pk, oh, U_pk, wuv4, wrr, r2e32, a14, a24,
                       l14, l24, att24, k4, br4, ba14, bl14)
    return out_pk.reshape(B, D)
